# baseline (speedup 1.0000x reference)
import jax
import jax.numpy as jnp
from jax import lax
from jax.experimental import pallas as pl
from jax.experimental.pallas import tpu as pltpu
from jax.experimental.pallas import tpu_sc as plsc

N = 10000
NP = 10240
F = 128
E = 320000
NC = 2
NS = 16
NW = NC * NS
EPT = 10240
EP = NW * EPT
ROWS_PER_TILE = NP // NS

CA = 80
NCHA = EPT // CA
CPQ = 32
NBUF = 4

CD = 128
NCHD = EPT // CD
HD = NCHD // 2


def _sc_mesh():
    return plsc.VectorSubcoreMesh(core_axis_name="c", subcore_axis_name="s",
                                  num_cores=NC, num_subcores=NS)


def _zero_fill(buf, rows):
    zeros = jnp.zeros((16,), jnp.float32)

    def zero_row(r, carry):
        for j in range(F // 16):
            buf[r, pl.ds(j * 16, 16)] = zeros
        return carry

    lax.fori_loop(0, rows, zero_row, 0)


def _zero_acc_slice(buf, acc_sh, sid, rows):
    base = sid * ROWS_PER_TILE
    for i in range(ROWS_PER_TILE // rows):
        pltpu.sync_copy(buf, acc_sh.at[pl.ds(base + i * rows, rows)])


def _writeback(acc_sh, out_hbm, cid, sid, buf_a, buf_b, sem_a, sem_b, rows):
    base = sid * ROWS_PER_TILE
    nch = ROWS_PER_TILE // rows
    bufs = (buf_a, buf_b)
    sems = (sem_a, sem_b)
    dr = [None, None]
    dw = [None, None]
    dr[0] = pltpu.async_copy(acc_sh.at[pl.ds(base, rows)], bufs[0], sems[0])
    for i in range(nch):
        b = i % 2
        dr[b].wait()
        dw[b] = pltpu.async_copy(
            bufs[b], out_hbm.at[cid, pl.ds(base + i * rows, rows)], sems[b])
        if i + 1 < nch:
            nb = (i + 1) % 2
            if dw[nb] is not None:
                dw[nb].wait()
            dr[nb] = pltpu.async_copy(
                acc_sh.at[pl.ds(base + (i + 1) * rows, rows)], bufs[nb],
                sems[nb])
    for b in range(2):
        if dw[b] is not None:
            dw[b].wait()


def _deg_body(dst_hbm, deg_out, dst_v, buf, one_v, sem_i, sem_s, sem_w, deg_sh):
    cid = lax.axis_index("c")
    sid = lax.axis_index("s")
    wid = cid * NS + sid

    _zero_fill(buf, CD)
    _zero_acc_slice(buf, deg_sh, sid, CD)

    ones = jnp.ones((16,), jnp.float32)

    def ones_row(r, carry):
        for j in range(F // 16):
            one_v[r, pl.ds(j * 16, 16)] = ones
        return carry

    lax.fori_loop(0, CD, ones_row, 0)
    plsc.subcore_barrier()

    for p in range(2):
        pltpu.async_copy(dst_hbm.at[wid, pl.ds(p * HD, HD)], dst_v,
                         sem_i).wait()
        pend = []
        for k in range(HD):
            if len(pend) >= 3:
                pend.pop(0).wait()
            pend.append(pltpu.async_copy(one_v, deg_sh.at[dst_v.at[k]],
                                         sem_s, add=True))
        for d in pend:
            d.wait()
    plsc.subcore_barrier()

    _writeback(deg_sh, deg_out, cid, sid, buf, one_v, sem_s, sem_w, CD)


def _deg_call(dst3):
    return pl.kernel(
        _deg_body,
        out_type=jax.ShapeDtypeStruct((NC, NP, F), jnp.float32),
        mesh=_sc_mesh(),
        scratch_types=[
            pltpu.VMEM((HD, CD), jnp.int32),
            pltpu.VMEM((CD, F), jnp.float32),
            pltpu.VMEM((CD, F), jnp.float32),
            pltpu.SemaphoreType.DMA,
            pltpu.SemaphoreType.DMA,
            pltpu.SemaphoreType.DMA,
            pltpu.VMEM_SHARED((NP, F), jnp.float32),
        ],
    )(dst3)


def _agg_body(g_hbm, src_hbm, dst_hbm, acc_out, src_v, dst_v,
              b0, b1, b2, b3, sem_i, sg0, sg1, sg2, sg3, ss0, ss1, ss2, ss3,
              acc_sh):
    cid = lax.axis_index("c")
    sid = lax.axis_index("s")
    wid = cid * NS + sid
    bufs = (b0, b1, b2, b3)
    sgs = (sg0, sg1, sg2, sg3)
    sss = (ss0, ss1, ss2, ss3)

    _zero_fill(b0, CA)
    _zero_acc_slice(b0, acc_sh, sid, CA)
    plsc.subcore_barrier()

    for q in range(NCHA // CPQ):
        pltpu.async_copy(src_hbm.at[wid, pl.ds(q * CPQ, CPQ)], src_v,
                         sem_i).wait()
        pltpu.async_copy(dst_hbm.at[wid, pl.ds(q * CPQ, CPQ)], dst_v,
                         sem_i).wait()
        dg = [None] * NBUF
        ds = [None] * NBUF
        dg[0] = pltpu.async_copy(g_hbm.at[src_v.at[0]], bufs[0], sgs[0])
        dg[1] = pltpu.async_copy(g_hbm.at[src_v.at[1]], bufs[1], sgs[1])
        for k in range(CPQ):
            b = k % NBUF
            dg[b].wait()
            ds[b] = pltpu.async_copy(bufs[b], acc_sh.at[dst_v.at[k]], sss[b],
                                     add=True)
            j = k + 2
            if j < CPQ:
                bj = j % NBUF
                if ds[bj] is not None:
                    ds[bj].wait()
                    ds[bj] = None
                dg[bj] = pltpu.async_copy(g_hbm.at[src_v.at[j]], bufs[bj],
                                          sgs[bj])
        for b in range(NBUF):
            if ds[b] is not None:
                ds[b].wait()
    plsc.subcore_barrier()

    _writeback(acc_sh, acc_out, cid, sid, b0, b1, sgs[0], sgs[1], CA)


def _agg_call(g, src3, dst3):
    return pl.kernel(
        _agg_body,
        out_type=jax.ShapeDtypeStruct((NC, NP, F), jnp.float32),
        mesh=_sc_mesh(),
        scratch_types=[
            pltpu.VMEM((CPQ, CA), jnp.int32),
            pltpu.VMEM((CPQ, CA), jnp.int32),
            pltpu.VMEM((CA, F), jnp.float32),
            pltpu.VMEM((CA, F), jnp.float32),
            pltpu.VMEM((CA, F), jnp.float32),
            pltpu.VMEM((CA, F), jnp.float32),
            pltpu.SemaphoreType.DMA,
            pltpu.SemaphoreType.DMA,
            pltpu.SemaphoreType.DMA,
            pltpu.SemaphoreType.DMA,
            pltpu.SemaphoreType.DMA,
            pltpu.SemaphoreType.DMA,
            pltpu.SemaphoreType.DMA,
            pltpu.SemaphoreType.DMA,
            pltpu.SemaphoreType.DMA,
            pltpu.VMEM_SHARED((NP, F), jnp.float32),
        ],
    )(g, src3, dst3)


_BLK = 640
_GRID = NP // _BLK
_BLK5 = 400
_GRID5 = N // _BLK5


def _k1_body(x_ref, w_ref, deg_ref, g_ref, dis_ref):
    dis = lax.rsqrt(deg_ref[0] + deg_ref[1] + 1.0)
    h = jnp.dot(x_ref[...], w_ref[...], preferred_element_type=jnp.float32)
    dis_ref[...] = dis
    g_ref[...] = h * dis


def _k1_call(x_p, W1, degs):
    return pl.pallas_call(
        _k1_body,
        out_shape=(jax.ShapeDtypeStruct((NP, F), jnp.float32),
                   jax.ShapeDtypeStruct((NP, F), jnp.float32)),
        grid=(_GRID,),
        in_specs=[
            pl.BlockSpec((_BLK, F), lambda i: (i, 0)),
            pl.BlockSpec((F, F), lambda i: (0, 0)),
            pl.BlockSpec((NC, _BLK, F), lambda i: (0, i, 0)),
        ],
        out_specs=(pl.BlockSpec((_BLK, F), lambda i: (i, 0)),
                   pl.BlockSpec((_BLK, F), lambda i: (i, 0))),
    )(x_p, W1, degs)


def _k3_body(acc_ref, g1_ref, dis_ref, b1_ref, w2_ref, g2_ref):
    dis = dis_ref[...]
    s = acc_ref[0] + acc_ref[1] + g1_ref[...]
    o1 = jnp.maximum(s * dis + b1_ref[...], 0.0)
    h2 = jnp.dot(o1, w2_ref[...], preferred_element_type=jnp.float32)
    g2_ref[...] = h2 * dis


def _k3_call(acc1, g1, dis2d, b1_2d, W2):
    return pl.pallas_call(
        _k3_body,
        out_shape=jax.ShapeDtypeStruct((NP, F), jnp.float32),
        grid=(_GRID,),
        in_specs=[
            pl.BlockSpec((NC, _BLK, F), lambda i: (0, i, 0)),
            pl.BlockSpec((_BLK, F), lambda i: (i, 0)),
            pl.BlockSpec((_BLK, F), lambda i: (i, 0)),
            pl.BlockSpec((1, F), lambda i: (0, 0)),
            pl.BlockSpec((F, F), lambda i: (0, 0)),
        ],
        out_specs=pl.BlockSpec((_BLK, F), lambda i: (i, 0)),
    )(acc1, g1, dis2d, b1_2d, W2)


def _k5_body(acc_ref, g2_ref, dis_ref, b2_ref, out_ref):
    z = (acc_ref[0] + acc_ref[1] + g2_ref[...]) * dis_ref[...] + b2_ref[...]
    out_ref[...] = 1.0 / (1.0 + jnp.exp(-z))


def _k5_call(acc2, g2, dis2d, b2_2d):
    return pl.pallas_call(
        _k5_body,
        out_shape=jax.ShapeDtypeStruct((N, F), jnp.float32),
        grid=(_GRID5,),
        in_specs=[
            pl.BlockSpec((NC, _BLK5, F), lambda i: (0, i, 0)),
            pl.BlockSpec((_BLK5, F), lambda i: (i, 0)),
            pl.BlockSpec((_BLK5, F), lambda i: (i, 0)),
            pl.BlockSpec((1, F), lambda i: (0, 0)),
        ],
        out_specs=pl.BlockSpec((_BLK5, F), lambda i: (i, 0)),
    )(acc2, g2, dis2d, b2_2d)


def kernel(x, edge_index, batch, W1, b1, W2, b2):
    src = edge_index[0].astype(jnp.int32)
    dst = edge_index[1].astype(jnp.int32)
    pad = N + (jnp.arange(EP - E, dtype=jnp.int32) % (NP - N))
    src_f = jnp.concatenate([src, pad])
    dst_f = jnp.concatenate([dst, pad])
    srcA = src_f.reshape(NW, NCHA, CA)
    dstA = dst_f.reshape(NW, NCHA, CA)
    dstD = dst_f.reshape(NW, NCHD, CD)
    x_p = jnp.zeros((NP, F), jnp.float32).at[:N].set(x)

    degs = _deg_call(dstD)
    g1, dis2d = _k1_call(x_p, W1, degs)
    acc1 = _agg_call(g1, srcA, dstA)
    g2 = _k3_call(acc1, g1, dis2d, b1.reshape(1, F), W2)
    acc2 = _agg_call(g2, srcA, dstA)
    return _k5_call(acc2, g2, dis2d, b2.reshape(1, F))


# speedup vs baseline: 1.0753x; 1.0753x over previous
import jax
import jax.numpy as jnp
from jax import lax
from jax.experimental import pallas as pl
from jax.experimental.pallas import tpu as pltpu
from jax.experimental.pallas import tpu_sc as plsc

N = 10000
NP = 10240
F = 128
E = 320000
NC = 2
NS = 16
NW = NC * NS
EPT = 10240
EP = NW * EPT
ROWS_PER_TILE = NP // NS

CA = 128
NCHA = EPT // CA
HA = NCHA // 2


def _sc_mesh():
    return plsc.VectorSubcoreMesh(core_axis_name="c", subcore_axis_name="s",
                                  num_cores=NC, num_subcores=NS)


def _zero_fill(buf, rows):
    zeros = jnp.zeros((16,), jnp.float32)

    def zero_row(r, carry):
        for j in range(F // 16):
            buf[r, pl.ds(j * 16, 16)] = zeros
        return carry

    lax.fori_loop(0, rows, zero_row, 0)


def _zero_acc_slice(buf, acc_sh, sid, rows):
    base = sid * ROWS_PER_TILE
    for i in range(ROWS_PER_TILE // rows):
        pltpu.sync_copy(buf, acc_sh.at[pl.ds(base + i * rows, rows)])


def _writeback(acc_sh, out_hbm, cid, sid, buf_a, buf_b, sem_a, sem_b, rows):
    base = sid * ROWS_PER_TILE
    nch = ROWS_PER_TILE // rows
    bufs = (buf_a, buf_b)
    sems = (sem_a, sem_b)
    dr = [None, None]
    dw = [None, None]
    dr[0] = pltpu.async_copy(acc_sh.at[pl.ds(base, rows)], bufs[0], sems[0])
    for i in range(nch):
        b = i % 2
        dr[b].wait()
        dw[b] = pltpu.async_copy(
            bufs[b], out_hbm.at[cid, pl.ds(base + i * rows, rows)], sems[b])
        if i + 1 < nch:
            nb = (i + 1) % 2
            if dw[nb] is not None:
                dw[nb].wait()
            dr[nb] = pltpu.async_copy(
                acc_sh.at[pl.ds(base + (i + 1) * rows, rows)], bufs[nb],
                sems[nb])
    for b in range(2):
        if dw[b] is not None:
            dw[b].wait()


def _deg_body(dst_hbm, deg_out, dst_v, buf, one_v, sem_i, sem_s, sem_w, deg_sh):
    cid = lax.axis_index("c")
    sid = lax.axis_index("s")
    wid = cid * NS + sid

    _zero_fill(buf, CA)
    _zero_acc_slice(buf, deg_sh, sid, CA)

    ones = jnp.ones((16,), jnp.float32)

    def ones_row(r, carry):
        for j in range(F // 16):
            one_v[r, pl.ds(j * 16, 16)] = ones
        return carry

    lax.fori_loop(0, CA, ones_row, 0)
    plsc.subcore_barrier()

    for p in range(2):
        pltpu.async_copy(dst_hbm.at[wid, pl.ds(p * HA, HA)], dst_v,
                         sem_i).wait()
        pend = []
        for k in range(HA):
            if len(pend) >= 3:
                pend.pop(0).wait()
            pend.append(pltpu.async_copy(one_v, deg_sh.at[dst_v.at[k]],
                                         sem_s, add=True))
        for d in pend:
            d.wait()
    plsc.subcore_barrier()

    _writeback(deg_sh, deg_out, cid, sid, buf, one_v, sem_s, sem_w, CA)


def _deg_call(dst3):
    return pl.kernel(
        _deg_body,
        out_type=jax.ShapeDtypeStruct((NC, NP, F), jnp.float32),
        mesh=_sc_mesh(),
        scratch_types=[
            pltpu.VMEM((HA, CA), jnp.int32),
            pltpu.VMEM((CA, F), jnp.float32),
            pltpu.VMEM((CA, F), jnp.float32),
            pltpu.SemaphoreType.DMA,
            pltpu.SemaphoreType.DMA,
            pltpu.SemaphoreType.DMA,
            pltpu.VMEM_SHARED((NP, F), jnp.float32),
        ],
    )(dst3)


def _agg_body(g_hbm, src_hbm, dst_hbm, acc_out, src_v, dst_v, b0, b1,
              sem_i, sg0, sg1, acc_sh):
    cid = lax.axis_index("c")
    sid = lax.axis_index("s")
    wid = cid * NS + sid

    _zero_fill(b0, CA)
    _zero_acc_slice(b0, acc_sh, sid, CA)
    plsc.subcore_barrier()

    for p in range(2):
        pltpu.async_copy(src_hbm.at[wid, pl.ds(p * HA, HA)], src_v,
                         sem_i).wait()
        pltpu.async_copy(dst_hbm.at[wid, pl.ds(p * HA, HA)], dst_v,
                         sem_i).wait()
        d0 = pltpu.async_copy(g_hbm.at[src_v.at[0]], b0, sg0)
        d1 = pltpu.async_copy(g_hbm.at[src_v.at[1]], b1, sg1)
        for k in range(HA):
            buf, sem, d = (b0, sg0, d0) if k % 2 == 0 else (b1, sg1, d1)
            d.wait()
            pltpu.sync_copy(buf, acc_sh.at[dst_v.at[k]], add=True)
            if k + 2 < HA:
                d_new = pltpu.async_copy(g_hbm.at[src_v.at[k + 2]], buf, sem)
                if k % 2 == 0:
                    d0 = d_new
                else:
                    d1 = d_new
    plsc.subcore_barrier()

    _writeback(acc_sh, acc_out, cid, sid, b0, b1, sg0, sg1, CA)


def _agg_call(g, src3, dst3):
    return pl.kernel(
        _agg_body,
        out_type=jax.ShapeDtypeStruct((NC, NP, F), jnp.float32),
        mesh=_sc_mesh(),
        scratch_types=[
            pltpu.VMEM((HA, CA), jnp.int32),
            pltpu.VMEM((HA, CA), jnp.int32),
            pltpu.VMEM((CA, F), jnp.float32),
            pltpu.VMEM((CA, F), jnp.float32),
            pltpu.SemaphoreType.DMA,
            pltpu.SemaphoreType.DMA,
            pltpu.SemaphoreType.DMA,
            pltpu.VMEM_SHARED((NP, F), jnp.float32),
        ],
    )(g, src3, dst3)


_BLK = 640
_GRID = NP // _BLK
_BLK5 = 400
_GRID5 = N // _BLK5


def _k1_body(x_ref, w_ref, deg_ref, g_ref, dis_ref):
    dis = lax.rsqrt(deg_ref[0] + deg_ref[1] + 1.0)
    h = jnp.dot(x_ref[...], w_ref[...], preferred_element_type=jnp.float32)
    dis_ref[...] = dis
    g_ref[...] = h * dis


def _k1_call(x_p, W1, degs):
    return pl.pallas_call(
        _k1_body,
        out_shape=(jax.ShapeDtypeStruct((NP, F), jnp.float32),
                   jax.ShapeDtypeStruct((NP, F), jnp.float32)),
        grid=(_GRID,),
        in_specs=[
            pl.BlockSpec((_BLK, F), lambda i: (i, 0)),
            pl.BlockSpec((F, F), lambda i: (0, 0)),
            pl.BlockSpec((NC, _BLK, F), lambda i: (0, i, 0)),
        ],
        out_specs=(pl.BlockSpec((_BLK, F), lambda i: (i, 0)),
                   pl.BlockSpec((_BLK, F), lambda i: (i, 0))),
    )(x_p, W1, degs)


def _k3_body(acc_ref, g1_ref, dis_ref, b1_ref, w2_ref, g2_ref):
    dis = dis_ref[...]
    s = acc_ref[0] + acc_ref[1] + g1_ref[...]
    o1 = jnp.maximum(s * dis + b1_ref[...], 0.0)
    h2 = jnp.dot(o1, w2_ref[...], preferred_element_type=jnp.float32)
    g2_ref[...] = h2 * dis


def _k3_call(acc1, g1, dis2d, b1_2d, W2):
    return pl.pallas_call(
        _k3_body,
        out_shape=jax.ShapeDtypeStruct((NP, F), jnp.float32),
        grid=(_GRID,),
        in_specs=[
            pl.BlockSpec((NC, _BLK, F), lambda i: (0, i, 0)),
            pl.BlockSpec((_BLK, F), lambda i: (i, 0)),
            pl.BlockSpec((_BLK, F), lambda i: (i, 0)),
            pl.BlockSpec((1, F), lambda i: (0, 0)),
            pl.BlockSpec((F, F), lambda i: (0, 0)),
        ],
        out_specs=pl.BlockSpec((_BLK, F), lambda i: (i, 0)),
    )(acc1, g1, dis2d, b1_2d, W2)


def _k5_body(acc_ref, g2_ref, dis_ref, b2_ref, out_ref):
    z = (acc_ref[0] + acc_ref[1] + g2_ref[...]) * dis_ref[...] + b2_ref[...]
    out_ref[...] = 1.0 / (1.0 + jnp.exp(-z))


def _k5_call(acc2, g2, dis2d, b2_2d):
    return pl.pallas_call(
        _k5_body,
        out_shape=jax.ShapeDtypeStruct((N, F), jnp.float32),
        grid=(_GRID5,),
        in_specs=[
            pl.BlockSpec((NC, _BLK5, F), lambda i: (0, i, 0)),
            pl.BlockSpec((_BLK5, F), lambda i: (i, 0)),
            pl.BlockSpec((_BLK5, F), lambda i: (i, 0)),
            pl.BlockSpec((1, F), lambda i: (0, 0)),
        ],
        out_specs=pl.BlockSpec((_BLK5, F), lambda i: (i, 0)),
    )(acc2, g2, dis2d, b2_2d)


def kernel(x, edge_index, batch, W1, b1, W2, b2):
    src = edge_index[0].astype(jnp.int32)
    dst = edge_index[1].astype(jnp.int32)
    pad = N + (jnp.arange(EP - E, dtype=jnp.int32) % (NP - N))
    src_f = jnp.concatenate([src, pad])
    dst_f = jnp.concatenate([dst, pad])
    srcA = src_f.reshape(NW, NCHA, CA)
    dstA = dst_f.reshape(NW, NCHA, CA)
    x_p = jnp.zeros((NP, F), jnp.float32).at[:N].set(x)

    degs = _deg_call(dstA)
    g1, dis2d = _k1_call(x_p, W1, degs)
    acc1 = _agg_call(g1, srcA, dstA)
    g2 = _k3_call(acc1, g1, dis2d, b1.reshape(1, F), W2)
    acc2 = _agg_call(g2, srcA, dstA)
    return _k5_call(acc2, g2, dis2d, b2.reshape(1, F))


# speedup vs baseline: 1.0757x; 1.0004x over previous
import jax
import jax.numpy as jnp
from jax import lax
from jax.experimental import pallas as pl
from jax.experimental.pallas import tpu as pltpu
from jax.experimental.pallas import tpu_sc as plsc

N = 10000
NP = 10240
F = 128
E = 320000
NC = 2
NS = 16
NW = NC * NS
EPT = 10240
EP = NW * EPT
ROWS_PER_TILE = NP // NS

CA = 128
NCHA = EPT // CA
HA = NCHA // 2


def _sc_mesh():
    return plsc.VectorSubcoreMesh(core_axis_name="c", subcore_axis_name="s",
                                  num_cores=NC, num_subcores=NS)


def _zero_fill(buf, rows):
    zeros = jnp.zeros((16,), jnp.float32)

    def zero_row(r, carry):
        for j in range(F // 16):
            buf[r, pl.ds(j * 16, 16)] = zeros
        return carry

    lax.fori_loop(0, rows, zero_row, 0)


def _zero_acc_slice(buf, acc_sh, sid, rows):
    base = sid * ROWS_PER_TILE
    for i in range(ROWS_PER_TILE // rows):
        pltpu.sync_copy(buf, acc_sh.at[pl.ds(base + i * rows, rows)])


def _writeback(acc_sh, out_hbm, cid, sid, buf_a, buf_b, sem_a, sem_b, rows):
    base = sid * ROWS_PER_TILE
    nch = ROWS_PER_TILE // rows
    bufs = (buf_a, buf_b)
    sems = (sem_a, sem_b)
    dr = [None, None]
    dw = [None, None]
    dr[0] = pltpu.async_copy(acc_sh.at[pl.ds(base, rows)], bufs[0], sems[0])
    for i in range(nch):
        b = i % 2
        dr[b].wait()
        dw[b] = pltpu.async_copy(
            bufs[b], out_hbm.at[cid, pl.ds(base + i * rows, rows)], sems[b])
        if i + 1 < nch:
            nb = (i + 1) % 2
            if dw[nb] is not None:
                dw[nb].wait()
            dr[nb] = pltpu.async_copy(
                acc_sh.at[pl.ds(base + (i + 1) * rows, rows)], bufs[nb],
                sems[nb])
    for b in range(2):
        if dw[b] is not None:
            dw[b].wait()


def _deg_body(dst_hbm, deg_out, dst_v, buf, one_v, sem_i, sem_s, sem_w, deg_sh):
    cid = lax.axis_index("c")
    sid = lax.axis_index("s")
    wid = cid * NS + sid

    _zero_fill(buf, CA)
    _zero_acc_slice(buf, deg_sh, sid, CA)

    ones = jnp.ones((16,), jnp.float32)

    def ones_row(r, carry):
        for j in range(F // 16):
            one_v[r, pl.ds(j * 16, 16)] = ones
        return carry

    lax.fori_loop(0, CA, ones_row, 0)
    plsc.subcore_barrier()

    for p in range(2):
        pltpu.async_copy(dst_hbm.at[wid, pl.ds(p * HA, HA)], dst_v,
                         sem_i).wait()
        pend = []
        for k in range(HA):
            if len(pend) >= 3:
                pend.pop(0).wait()
            pend.append(pltpu.async_copy(one_v, deg_sh.at[dst_v.at[k]],
                                         sem_s, add=True))
        for d in pend:
            d.wait()
    plsc.subcore_barrier()

    _writeback(deg_sh, deg_out, cid, sid, buf, one_v, sem_s, sem_w, CA)


def _deg_call(dst3):
    return pl.kernel(
        _deg_body,
        out_type=jax.ShapeDtypeStruct((NC, NP, F), jnp.float32),
        mesh=_sc_mesh(),
        scratch_types=[
            pltpu.VMEM((HA, CA), jnp.int32),
            pltpu.VMEM((CA, F), jnp.float32),
            pltpu.VMEM((CA, F), jnp.float32),
            pltpu.SemaphoreType.DMA,
            pltpu.SemaphoreType.DMA,
            pltpu.SemaphoreType.DMA,
            pltpu.VMEM_SHARED((NP, F), jnp.float32),
        ],
    )(dst3)


def _agg_body(g_hbm, src_hbm, dst_hbm, acc_out, src_v, dst_v, b0, b1,
              sem_i, sg0, sg1, acc_sh):
    cid = lax.axis_index("c")
    sid = lax.axis_index("s")
    wid = cid * NS + sid

    _zero_fill(b0, CA)
    _zero_acc_slice(b0, acc_sh, sid, CA)
    plsc.subcore_barrier()

    for p in range(2):
        pltpu.async_copy(src_hbm.at[wid, pl.ds(p * HA, HA)], src_v,
                         sem_i).wait()
        pltpu.async_copy(dst_hbm.at[wid, pl.ds(p * HA, HA)], dst_v,
                         sem_i).wait()
        d0 = pltpu.async_copy(g_hbm.at[src_v.at[0]], b0, sg0)
        d1 = pltpu.async_copy(g_hbm.at[src_v.at[1]], b1, sg1)
        for k in range(HA):
            buf, sem, d = (b0, sg0, d0) if k % 2 == 0 else (b1, sg1, d1)
            d.wait()
            pltpu.sync_copy(buf, acc_sh.at[dst_v.at[k]], add=True)
            if k + 2 < HA:
                d_new = pltpu.async_copy(g_hbm.at[src_v.at[k + 2]], buf, sem)
                if k % 2 == 0:
                    d0 = d_new
                else:
                    d1 = d_new
    plsc.subcore_barrier()

    _writeback(acc_sh, acc_out, cid, sid, b0, b1, sg0, sg1, CA)


def _agg_call(g, src3, dst3):
    return pl.kernel(
        _agg_body,
        out_type=jax.ShapeDtypeStruct((NC, NP, F), jnp.float32),
        mesh=_sc_mesh(),
        scratch_types=[
            pltpu.VMEM((HA, CA), jnp.int32),
            pltpu.VMEM((HA, CA), jnp.int32),
            pltpu.VMEM((CA, F), jnp.float32),
            pltpu.VMEM((CA, F), jnp.float32),
            pltpu.SemaphoreType.DMA,
            pltpu.SemaphoreType.DMA,
            pltpu.SemaphoreType.DMA,
            pltpu.VMEM_SHARED((NP, F), jnp.float32),
        ],
    )(g, src3, dst3)


_BLK = 640
_GRID = NP // _BLK
_BLK5 = 400
_GRID5 = N // _BLK5


def _k1a_body(x_ref, w_ref, h_ref):
    h_ref[...] = jnp.dot(x_ref[...], w_ref[...],
                         preferred_element_type=jnp.float32)


def _k1a_call(x, W1):
    return pl.pallas_call(
        _k1a_body,
        out_shape=jax.ShapeDtypeStruct((NP, F), jnp.float32),
        grid=(_GRID,),
        in_specs=[
            pl.BlockSpec((_BLK, F), lambda i: (i, 0)),
            pl.BlockSpec((F, F), lambda i: (0, 0)),
        ],
        out_specs=pl.BlockSpec((_BLK, F), lambda i: (i, 0)),
    )(x, W1)


def _k1b_body(h_ref, deg_ref, g_ref, dis_ref):
    dis = lax.rsqrt(deg_ref[0] + deg_ref[1] + 1.0)
    dis_ref[...] = dis
    g_ref[...] = h_ref[...] * dis


def _k1b_call(h1, degs):
    return pl.pallas_call(
        _k1b_body,
        out_shape=(jax.ShapeDtypeStruct((NP, F), jnp.float32),
                   jax.ShapeDtypeStruct((NP, F), jnp.float32)),
        grid=(_GRID,),
        in_specs=[
            pl.BlockSpec((_BLK, F), lambda i: (i, 0)),
            pl.BlockSpec((NC, _BLK, F), lambda i: (0, i, 0)),
        ],
        out_specs=(pl.BlockSpec((_BLK, F), lambda i: (i, 0)),
                   pl.BlockSpec((_BLK, F), lambda i: (i, 0))),
    )(h1, degs)


def _k3_body(acc_ref, g1_ref, dis_ref, b1_ref, w2_ref, g2_ref):
    dis = dis_ref[...]
    s = acc_ref[0] + acc_ref[1] + g1_ref[...]
    o1 = jnp.maximum(s * dis + b1_ref[...], 0.0)
    h2 = jnp.dot(o1, w2_ref[...], preferred_element_type=jnp.float32)
    g2_ref[...] = h2 * dis


def _k3_call(acc1, g1, dis2d, b1_2d, W2):
    return pl.pallas_call(
        _k3_body,
        out_shape=jax.ShapeDtypeStruct((NP, F), jnp.float32),
        grid=(_GRID,),
        in_specs=[
            pl.BlockSpec((NC, _BLK, F), lambda i: (0, i, 0)),
            pl.BlockSpec((_BLK, F), lambda i: (i, 0)),
            pl.BlockSpec((_BLK, F), lambda i: (i, 0)),
            pl.BlockSpec((1, F), lambda i: (0, 0)),
            pl.BlockSpec((F, F), lambda i: (0, 0)),
        ],
        out_specs=pl.BlockSpec((_BLK, F), lambda i: (i, 0)),
    )(acc1, g1, dis2d, b1_2d, W2)


def _k5_body(acc_ref, g2_ref, dis_ref, b2_ref, out_ref):
    z = (acc_ref[0] + acc_ref[1] + g2_ref[...]) * dis_ref[...] + b2_ref[...]
    out_ref[...] = 1.0 / (1.0 + jnp.exp(-z))


def _k5_call(acc2, g2, dis2d, b2_2d):
    return pl.pallas_call(
        _k5_body,
        out_shape=jax.ShapeDtypeStruct((N, F), jnp.float32),
        grid=(_GRID5,),
        in_specs=[
            pl.BlockSpec((NC, _BLK5, F), lambda i: (0, i, 0)),
            pl.BlockSpec((_BLK5, F), lambda i: (i, 0)),
            pl.BlockSpec((_BLK5, F), lambda i: (i, 0)),
            pl.BlockSpec((1, F), lambda i: (0, 0)),
        ],
        out_specs=pl.BlockSpec((_BLK5, F), lambda i: (i, 0)),
    )(acc2, g2, dis2d, b2_2d)


def kernel(x, edge_index, batch, W1, b1, W2, b2):
    src = edge_index[0].astype(jnp.int32)
    dst = edge_index[1].astype(jnp.int32)
    pad = N + (jnp.arange(EP - E, dtype=jnp.int32) % (NP - N))
    srcA = jnp.concatenate([src, pad]).reshape(NW, NCHA, CA)
    dstA = jnp.concatenate([dst, pad]).reshape(NW, NCHA, CA)

    h1 = _k1a_call(x, W1)
    degs = _deg_call(dstA)
    g1, dis2d = _k1b_call(h1, degs)
    acc1 = _agg_call(g1, srcA, dstA)
    g2 = _k3_call(acc1, g1, dis2d, b1.reshape(1, F), W2)
    acc2 = _agg_call(g2, srcA, dstA)
    return _k5_call(acc2, g2, dis2d, b2.reshape(1, F))
